# Initial kernel scaffold; baseline (speedup 1.0000x reference)
import jax
import jax.numpy as jnp
from jax.experimental import pallas as pl


def kernel(x_in_node, edge_idx, edge_in_attr, A_w, A_b, B_w, B_b, C_w, C_b, D_w, D_b, E_w, E_b, Rproj_e_w):
    raise NotImplementedError("write your pallas kernel here")


# speedup vs baseline: 1.0758x; 1.0758x over previous
import functools

import jax
import jax.numpy as jnp
from jax import lax
from jax.experimental import pallas as pl
from jax.experimental.pallas import tpu as pltpu
from jax.experimental.pallas import tpu_sc as plsc

_N = 10000
_E = 320000
_D = 128
_NPAD = 10112
_NTILES = 32
_EPT = _E // _NTILES
_CHUNK = 40
_NITER = _EPT // _CHUNK
_RSUB = _NPAD // 16


def _node_proj_body(x_ref, aw, ab, bw, bb, dw, db, ew, eb2, ax_ref, dx_ref, ebt_ref):
    f32 = jnp.float32
    x = x_ref[...]
    ax_ref[...] = jnp.dot(x, aw[...], preferred_element_type=f32) + ab[...]
    dx_ref[...] = jnp.dot(x, dw[...], preferred_element_type=f32) + db[...]
    ebt_ref[:, :_D] = jnp.dot(x, ew[...], preferred_element_type=f32) + eb2[...]
    ebt_ref[:, _D:] = jnp.dot(x, bw[...], preferred_element_type=f32) + bb[...]


def _edge_proj_body(attr_ref, cw, cb, rw, cr_ref):
    f32 = jnp.float32
    a = attr_ref[...]
    cr_ref[:, :_D] = jnp.dot(a, cw[...], preferred_element_type=f32) + cb[...]
    cr_ref[:, _D:] = jnp.dot(a, rw[...], preferred_element_type=f32)


def _tail_body(x_ref, ax_ref, ag_ref, out_ref):
    s = ax_ref[...] + ag_ref[0] + ag_ref[1]
    out_ref[...] = x_ref[...] + jnp.maximum(s, 0.0)


def _sc_edge_body(row3_hbm, col3_hbm, dx_hbm, eb_hbm, cr_hbm, zeros_hbm,
                  ef_hbm, aggr_hbm,
                  row_v, col_v, dxr, ebr, crr, efb, gb, aggr_sh,
                  sem1, sem2):
    c = lax.axis_index("c")
    s = lax.axis_index("s")
    wid = c * 16 + s
    rows0 = s * _RSUB
    pltpu.sync_copy(zeros_hbm.at[pl.ds(rows0, _RSUB)], aggr_sh.at[pl.ds(rows0, _RSUB)])
    plsc.subcore_barrier()

    base0 = wid * _EPT

    def chunk_body(i, carry):
        base = base0 + i * _CHUNK
        pltpu.sync_copy(row3_hbm.at[wid, i], row_v)
        pltpu.sync_copy(col3_hbm.at[wid, i], col_v)
        pltpu.async_copy(dx_hbm.at[row_v], dxr, sem1).wait()
        pltpu.async_copy(eb_hbm.at[col_v], ebr, sem2).wait()
        pltpu.sync_copy(cr_hbm.at[pl.ds(base, _CHUNK)], crr)

        def edge_row(r, carry2):
            for v in range(_D // 16):
                o = v * 16
                d = dxr[r, pl.ds(o, 16)]
                ex = ebr[r, pl.ds(o, 16)]
                bx = ebr[r, pl.ds(_D + o, 16)]
                ce = crr[r, pl.ds(o, 16)]
                rp = crr[r, pl.ds(_D + o, 16)]
                e = d + ex + ce
                efb[r, pl.ds(o, 16)] = jnp.maximum(e, 0.0) + rp
                gb[r, pl.ds(o, 16)] = bx / (1.0 + jnp.exp(-e))
            return carry2

        lax.fori_loop(0, _CHUNK, edge_row, 0)
        pltpu.sync_copy(efb, ef_hbm.at[pl.ds(base, _CHUNK)])
        pltpu.sync_copy(gb, aggr_sh.at[row_v], add=True)
        return carry

    lax.fori_loop(0, _NITER, chunk_body, 0)
    plsc.subcore_barrier()
    pltpu.sync_copy(aggr_sh.at[pl.ds(rows0, _RSUB)],
                    aggr_hbm.at[c, pl.ds(rows0, _RSUB)])


def kernel(x_in_node, edge_idx, edge_in_attr, A_w, A_b, B_w, B_b, C_w, C_b,
           D_w, D_b, E_w, E_b, Rproj_e_w):
    f32 = jnp.float32

    nb = 2000
    wspec = pl.BlockSpec((_D, _D), lambda i: (0, 0))
    bspec = pl.BlockSpec((_D,), lambda i: (0,))
    nspec = pl.BlockSpec((nb, _D), lambda i: (i, 0))
    ax, dx, ebt = pl.pallas_call(
        _node_proj_body,
        grid=(_N // nb,),
        in_specs=[nspec, wspec, bspec, wspec, bspec, wspec, bspec, wspec, bspec],
        out_specs=[nspec, nspec, pl.BlockSpec((nb, 2 * _D), lambda i: (i, 0))],
        out_shape=[
            jax.ShapeDtypeStruct((_N, _D), f32),
            jax.ShapeDtypeStruct((_N, _D), f32),
            jax.ShapeDtypeStruct((_N, 2 * _D), f32),
        ],
    )(x_in_node, A_w, A_b, B_w, B_b, D_w, D_b, E_w, E_b)

    ebk = 4000
    cr = pl.pallas_call(
        _edge_proj_body,
        grid=(_E // ebk,),
        in_specs=[
            pl.BlockSpec((ebk, 16), lambda i: (i, 0)),
            pl.BlockSpec((16, _D), lambda i: (0, 0)),
            pl.BlockSpec((_D,), lambda i: (0,)),
            pl.BlockSpec((16, _D), lambda i: (0, 0)),
        ],
        out_specs=pl.BlockSpec((ebk, 2 * _D), lambda i: (i, 0)),
        out_shape=jax.ShapeDtypeStruct((_E, 2 * _D), f32),
    )(edge_in_attr, C_w, C_b, Rproj_e_w)

    row3 = edge_idx[0].reshape(_NTILES, _NITER, _CHUNK)
    col3 = edge_idx[1].reshape(_NTILES, _NITER, _CHUNK)
    zeros = jnp.zeros((_NPAD, _D), f32)

    mesh = plsc.VectorSubcoreMesh(core_axis_name="c", subcore_axis_name="s")
    sc_call = functools.partial(
        pl.kernel,
        out_type=(
            jax.ShapeDtypeStruct((_E, _D), f32),
            jax.ShapeDtypeStruct((2, _NPAD, _D), f32),
        ),
        mesh=mesh,
        scratch_types=[
            pltpu.VMEM((_CHUNK,), jnp.int32),
            pltpu.VMEM((_CHUNK,), jnp.int32),
            pltpu.VMEM((_CHUNK, _D), f32),
            pltpu.VMEM((_CHUNK, 2 * _D), f32),
            pltpu.VMEM((_CHUNK, 2 * _D), f32),
            pltpu.VMEM((_CHUNK, _D), f32),
            pltpu.VMEM((_CHUNK, _D), f32),
            pltpu.VMEM_SHARED((_NPAD, _D), f32),
            pltpu.SemaphoreType.DMA,
            pltpu.SemaphoreType.DMA,
        ],
    )(_sc_edge_body)
    e_final, aggr = sc_call(row3, col3, dx, ebt, cr, zeros)

    nb2 = 400
    x_final = pl.pallas_call(
        _tail_body,
        grid=(_N // nb2,),
        in_specs=[
            pl.BlockSpec((nb2, _D), lambda i: (i, 0)),
            pl.BlockSpec((nb2, _D), lambda i: (i, 0)),
            pl.BlockSpec((2, nb2, _D), lambda i: (0, i, 0)),
        ],
        out_specs=pl.BlockSpec((nb2, _D), lambda i: (i, 0)),
        out_shape=jax.ShapeDtypeStruct((_N, _D), f32),
    )(x_in_node, ax, aggr)

    return (x_final, e_final)


# speedup vs baseline: 1.4013x; 1.3027x over previous
import functools

import jax
import jax.numpy as jnp
from jax import lax
from jax.experimental import pallas as pl
from jax.experimental.pallas import tpu as pltpu
from jax.experimental.pallas import tpu_sc as plsc

_N = 10000
_E = 320000
_D = 128
_NPAD = 10112
_NTILES = 32
_CHUNK = 32
_NCH = _E // _CHUNK
_NMAIN = 312
_NEXTRA = _NCH - _NMAIN * _NTILES
_RSUB = _NPAD // 16


def _node_proj_body(x_ref, aw, ab, bw, bb, dw, db, ew, eb2, ax_ref, dx_ref, ebt_ref):
    f32 = jnp.float32
    x = x_ref[...]
    ax_ref[...] = jnp.dot(x, aw[...], preferred_element_type=f32) + ab[...]
    dx_ref[...] = jnp.dot(x, dw[...], preferred_element_type=f32) + db[...]
    ebt_ref[:, :_D] = jnp.dot(x, ew[...], preferred_element_type=f32) + eb2[...]
    ebt_ref[:, _D:] = jnp.dot(x, bw[...], preferred_element_type=f32) + bb[...]


def _edge_proj_body(attr_ref, cw, cb, rw, cr_ref):
    f32 = jnp.float32
    a = attr_ref[...]
    cr_ref[:, :_D] = jnp.dot(a, cw[...], preferred_element_type=f32) + cb[...]
    cr_ref[:, _D:] = jnp.dot(a, rw[...], preferred_element_type=f32)


def _tail_body(x_ref, ax_ref, ag_ref, out_ref):
    s = ax_ref[...] + ag_ref[0] + ag_ref[1]
    out_ref[...] = x_ref[...] + jnp.maximum(s, 0.0)


def _sc_edge_body(rc_hbm, dx_hbm, eb_hbm, cr_hbm, zeros_hbm,
                  ef_hbm, aggr_hbm,
                  rc0, rc1, dxr0, dxr1, ebr0, ebr1, crr0, crr1, gb0, gb1,
                  sga0, sga1, sgb0, sgb1, scr0, scr1, sef0, sef1, ssc0, ssc1,
                  aggr_sh):
    c = lax.axis_index("c")
    s = lax.axis_index("s")
    wid = c * 16 + s

    slots = (
        (rc0, dxr0, ebr0, crr0, gb0, sga0, sgb0, scr0, sef0, ssc0),
        (rc1, dxr1, ebr1, crr1, gb1, sga1, sgb1, scr1, sef1, ssc1),
    )

    def load_inputs(slot, j):
        rc, dxr, ebr, crr, _, sga, sgb, scr, _, _ = slots[slot]
        pltpu.sync_copy(rc_hbm.at[j], rc)
        pltpu.async_copy(dx_hbm.at[rc.at[0]], dxr, sga)
        pltpu.async_copy(eb_hbm.at[rc.at[1]], ebr, sgb)
        pltpu.async_copy(cr_hbm.at[pl.ds(j * _CHUNK, _CHUNK)], crr, scr)

    def wait_inputs(slot):
        rc, dxr, ebr, crr, _, sga, sgb, scr, _, _ = slots[slot]
        pltpu.make_async_copy(dx_hbm.at[rc.at[0]], dxr, sga).wait()
        pltpu.make_async_copy(eb_hbm.at[rc.at[1]], ebr, sgb).wait()
        pltpu.make_async_copy(cr_hbm.at[pl.ds(0, _CHUNK)], crr, scr).wait()

    def compute(slot):
        _, dxr, ebr, crr, gb, _, _, _, _, _ = slots[slot]

        def edge_row(r, carry):
            for v in range(_D // 16):
                o = v * 16
                d = dxr[r, pl.ds(o, 16)]
                ex = ebr[r, pl.ds(o, 16)]
                bx = ebr[r, pl.ds(_D + o, 16)]
                ce = crr[r, pl.ds(o, 16)]
                rp = crr[r, pl.ds(_D + o, 16)]
                e = d + ex + ce
                dxr[r, pl.ds(o, 16)] = jnp.maximum(e, 0.0) + rp
                gb[r, pl.ds(o, 16)] = bx / (1.0 + jnp.exp(-e))
            return carry

        lax.fori_loop(0, _CHUNK, edge_row, 0)

    def store_outputs(slot, j):
        rc, dxr, _, _, gb, _, _, _, sef, ssc = slots[slot]
        pltpu.async_copy(dxr, ef_hbm.at[pl.ds(j * _CHUNK, _CHUNK)], sef)
        pltpu.async_copy(gb, aggr_sh.at[rc.at[0]], ssc, add=True)

    def wait_outputs(slot):
        rc, dxr, _, _, gb, _, _, _, sef, ssc = slots[slot]
        pltpu.make_async_copy(dxr, ef_hbm.at[pl.ds(0, _CHUNK)], sef).wait()
        pltpu.make_async_copy(gb, aggr_sh.at[rc.at[0]], ssc).wait()

    rows0 = s * _RSUB
    pltpu.sync_copy(zeros_hbm.at[pl.ds(rows0, _RSUB)], aggr_sh.at[pl.ds(rows0, _RSUB)])
    plsc.subcore_barrier()

    j0 = wid * _NMAIN
    load_inputs(0, j0)

    def pair_body(ii, carry):
        ja = j0 + 2 * ii

        @pl.when(ii > 0)
        def _():
            wait_outputs(1)

        load_inputs(1, ja + 1)
        wait_inputs(0)
        compute(0)
        store_outputs(0, ja)
        wait_inputs(1)
        compute(1)
        store_outputs(1, ja + 1)

        @pl.when(ii < _NMAIN // 2 - 1)
        def _():
            wait_outputs(0)
            load_inputs(0, ja + 2)

        return carry

    lax.fori_loop(0, _NMAIN // 2, pair_body, 0)
    wait_outputs(0)
    wait_outputs(1)

    @pl.when(wid < _NEXTRA)
    def _():
        je = _NTILES * _NMAIN + wid
        load_inputs(0, je)
        wait_inputs(0)
        compute(0)
        store_outputs(0, je)
        wait_outputs(0)

    plsc.subcore_barrier()
    pltpu.sync_copy(aggr_sh.at[pl.ds(rows0, _RSUB)],
                    aggr_hbm.at[c, pl.ds(rows0, _RSUB)])


def kernel(x_in_node, edge_idx, edge_in_attr, A_w, A_b, B_w, B_b, C_w, C_b,
           D_w, D_b, E_w, E_b, Rproj_e_w):
    f32 = jnp.float32

    nb = 2000
    wspec = pl.BlockSpec((_D, _D), lambda i: (0, 0))
    bspec = pl.BlockSpec((_D,), lambda i: (0,))
    nspec = pl.BlockSpec((nb, _D), lambda i: (i, 0))
    ax, dx, ebt = pl.pallas_call(
        _node_proj_body,
        grid=(_N // nb,),
        in_specs=[nspec, wspec, bspec, wspec, bspec, wspec, bspec, wspec, bspec],
        out_specs=[nspec, nspec, pl.BlockSpec((nb, 2 * _D), lambda i: (i, 0))],
        out_shape=[
            jax.ShapeDtypeStruct((_N, _D), f32),
            jax.ShapeDtypeStruct((_N, _D), f32),
            jax.ShapeDtypeStruct((_N, 2 * _D), f32),
        ],
    )(x_in_node, A_w, A_b, B_w, B_b, D_w, D_b, E_w, E_b)

    ebk = 4000
    cr = pl.pallas_call(
        _edge_proj_body,
        grid=(_E // ebk,),
        in_specs=[
            pl.BlockSpec((ebk, 16), lambda i: (i, 0)),
            pl.BlockSpec((16, _D), lambda i: (0, 0)),
            pl.BlockSpec((_D,), lambda i: (0,)),
            pl.BlockSpec((16, _D), lambda i: (0, 0)),
        ],
        out_specs=pl.BlockSpec((ebk, 2 * _D), lambda i: (i, 0)),
        out_shape=jax.ShapeDtypeStruct((_E, 2 * _D), f32),
    )(edge_in_attr, C_w, C_b, Rproj_e_w)

    rc = jnp.stack(
        [edge_idx[0].reshape(_NCH, _CHUNK), edge_idx[1].reshape(_NCH, _CHUNK)],
        axis=1,
    )
    zeros = jnp.zeros((_NPAD, _D), f32)

    mesh = plsc.VectorSubcoreMesh(core_axis_name="c", subcore_axis_name="s")
    sc_call = functools.partial(
        pl.kernel,
        out_type=(
            jax.ShapeDtypeStruct((_E, _D), f32),
            jax.ShapeDtypeStruct((2, _NPAD, _D), f32),
        ),
        mesh=mesh,
        scratch_types=[
            pltpu.VMEM((2, _CHUNK), jnp.int32),
            pltpu.VMEM((2, _CHUNK), jnp.int32),
            pltpu.VMEM((_CHUNK, _D), f32),
            pltpu.VMEM((_CHUNK, _D), f32),
            pltpu.VMEM((_CHUNK, 2 * _D), f32),
            pltpu.VMEM((_CHUNK, 2 * _D), f32),
            pltpu.VMEM((_CHUNK, 2 * _D), f32),
            pltpu.VMEM((_CHUNK, 2 * _D), f32),
            pltpu.VMEM((_CHUNK, _D), f32),
            pltpu.VMEM((_CHUNK, _D), f32),
            pltpu.SemaphoreType.DMA,
            pltpu.SemaphoreType.DMA,
            pltpu.SemaphoreType.DMA,
            pltpu.SemaphoreType.DMA,
            pltpu.SemaphoreType.DMA,
            pltpu.SemaphoreType.DMA,
            pltpu.SemaphoreType.DMA,
            pltpu.SemaphoreType.DMA,
            pltpu.SemaphoreType.DMA,
            pltpu.SemaphoreType.DMA,
            pltpu.VMEM_SHARED((_NPAD, _D), f32),
        ],
    )(_sc_edge_body)
    e_final, aggr = sc_call(rc, dx, ebt, cr, zeros)

    nb2 = 400
    x_final = pl.pallas_call(
        _tail_body,
        grid=(_N // nb2,),
        in_specs=[
            pl.BlockSpec((nb2, _D), lambda i: (i, 0)),
            pl.BlockSpec((nb2, _D), lambda i: (i, 0)),
            pl.BlockSpec((2, nb2, _D), lambda i: (0, i, 0)),
        ],
        out_specs=pl.BlockSpec((nb2, _D), lambda i: (i, 0)),
        out_shape=jax.ShapeDtypeStruct((_N, _D), f32),
    )(x_in_node, ax, aggr)

    return (x_final, e_final)


# speedup vs baseline: 3.6314x; 2.5913x over previous
import functools

import jax
import jax.numpy as jnp
from jax import lax
from jax.experimental import pallas as pl
from jax.experimental.pallas import tpu as pltpu
from jax.experimental.pallas import tpu_sc as plsc

_N = 10000
_E = 320000
_D = 128
_NPAD = 10112
_NTILES = 32
_CHUNK = 32
_NCH = _E // _CHUNK
_NMAIN = 312
_NEXTRA = _NCH - _NMAIN * _NTILES
_RSUB = _NPAD // 16


def _node_proj_body(x_ref, aw, ab, bw, bb, dw, db, ew, eb2, ax_ref, dx_ref, ebt_ref):
    f32 = jnp.float32
    x = x_ref[...]
    ax_ref[...] = jnp.dot(x, aw[...], preferred_element_type=f32) + ab[...]
    dx_ref[...] = jnp.dot(x, dw[...], preferred_element_type=f32) + db[...]
    ebt_ref[:, :_D] = jnp.dot(x, ew[...], preferred_element_type=f32) + eb2[...]
    ebt_ref[:, _D:] = jnp.dot(x, bw[...], preferred_element_type=f32) + bb[...]


def _edge_proj_body(attr_ref, cw, cb, rw, cr_ref):
    f32 = jnp.float32
    a = attr_ref[...]
    cr_ref[:, :_D] = jnp.dot(a, cw[...], preferred_element_type=f32) + cb[...]
    cr_ref[:, _D:] = jnp.dot(a, rw[...], preferred_element_type=f32)


def _tail_body(x_ref, ax_ref, ag_ref, out_ref):
    s = ax_ref[...] + ag_ref[0] + ag_ref[1]
    out_ref[...] = x_ref[...] + jnp.maximum(s, 0.0)


def _sc_edge_body(rc_hbm, dx_hbm, eb_hbm, cr_hbm, zeros_hbm,
                  ef_hbm, aggr_hbm,
                  rc0, rc1, dxr0, dxr1, ebr0, ebr1, crr0, crr1, gb0, gb1,
                  sga0, sga1, sgb0, sgb1, scr0, scr1, sef0, sef1, ssc0, ssc1,
                  aggr_sh):
    c = lax.axis_index("c")
    s = lax.axis_index("s")
    wid = c * 16 + s

    slots = (
        (rc0, dxr0, ebr0, crr0, gb0, sga0, sgb0, scr0, sef0, ssc0),
        (rc1, dxr1, ebr1, crr1, gb1, sga1, sgb1, scr1, sef1, ssc1),
    )

    def load_inputs(slot, j):
        rc, dxr, ebr, crr, _, sga, sgb, scr, _, _ = slots[slot]
        pltpu.sync_copy(rc_hbm.at[j], rc)
        pltpu.async_copy(dx_hbm.at[rc.at[0]], dxr, sga)
        pltpu.async_copy(eb_hbm.at[rc.at[1]], ebr, sgb)
        pltpu.async_copy(cr_hbm.at[pl.ds(j * _CHUNK, _CHUNK)], crr, scr)

    def wait_inputs(slot):
        rc, dxr, ebr, crr, _, sga, sgb, scr, _, _ = slots[slot]
        pltpu.make_async_copy(dx_hbm.at[rc.at[0]], dxr, sga).wait()
        pltpu.make_async_copy(eb_hbm.at[rc.at[1]], ebr, sgb).wait()
        pltpu.make_async_copy(cr_hbm.at[pl.ds(0, _CHUNK)], crr, scr).wait()

    def compute(slot):
        _, dxr, ebr, crr, gb, _, _, _, _, _ = slots[slot]

        @plsc.parallel_loop(0, _CHUNK * (_D // 16), unroll=8)
        def _vreg_body(j):
            r = j >> 3
            o = (j & 7) * 16
            d = dxr[r, pl.ds(o, 16)]
            ex = ebr[r, pl.ds(o, 16)]
            bx = ebr[r, pl.ds(_D + o, 16)]
            ce = crr[r, pl.ds(o, 16)]
            rp = crr[r, pl.ds(_D + o, 16)]
            e = d + ex + ce
            dxr[r, pl.ds(o, 16)] = jnp.maximum(e, 0.0) + rp
            gb[r, pl.ds(o, 16)] = bx / (1.0 + jnp.exp(-e))

    def store_outputs(slot, j):
        rc, dxr, _, _, gb, _, _, _, sef, ssc = slots[slot]
        pltpu.async_copy(dxr, ef_hbm.at[pl.ds(j * _CHUNK, _CHUNK)], sef)
        pltpu.async_copy(gb, aggr_sh.at[rc.at[0]], ssc, add=True)

    def wait_outputs(slot):
        rc, dxr, _, _, gb, _, _, _, sef, ssc = slots[slot]
        pltpu.make_async_copy(dxr, ef_hbm.at[pl.ds(0, _CHUNK)], sef).wait()
        pltpu.make_async_copy(gb, aggr_sh.at[rc.at[0]], ssc).wait()

    rows0 = s * _RSUB
    pltpu.sync_copy(zeros_hbm.at[pl.ds(rows0, _RSUB)], aggr_sh.at[pl.ds(rows0, _RSUB)])
    plsc.subcore_barrier()

    j0 = wid * _NMAIN
    load_inputs(0, j0)

    def pair_body(ii, carry):
        ja = j0 + 2 * ii

        @pl.when(ii > 0)
        def _():
            wait_outputs(1)

        load_inputs(1, ja + 1)
        wait_inputs(0)
        compute(0)
        store_outputs(0, ja)
        wait_inputs(1)
        compute(1)
        store_outputs(1, ja + 1)

        @pl.when(ii < _NMAIN // 2 - 1)
        def _():
            wait_outputs(0)
            load_inputs(0, ja + 2)

        return carry

    lax.fori_loop(0, _NMAIN // 2, pair_body, 0)
    wait_outputs(0)
    wait_outputs(1)

    @pl.when(wid < _NEXTRA)
    def _():
        je = _NTILES * _NMAIN + wid
        load_inputs(0, je)
        wait_inputs(0)
        compute(0)
        store_outputs(0, je)
        wait_outputs(0)

    plsc.subcore_barrier()
    pltpu.sync_copy(aggr_sh.at[pl.ds(rows0, _RSUB)],
                    aggr_hbm.at[c, pl.ds(rows0, _RSUB)])


def kernel(x_in_node, edge_idx, edge_in_attr, A_w, A_b, B_w, B_b, C_w, C_b,
           D_w, D_b, E_w, E_b, Rproj_e_w):
    f32 = jnp.float32

    nb = 2000
    wspec = pl.BlockSpec((_D, _D), lambda i: (0, 0))
    bspec = pl.BlockSpec((_D,), lambda i: (0,))
    nspec = pl.BlockSpec((nb, _D), lambda i: (i, 0))
    ax, dx, ebt = pl.pallas_call(
        _node_proj_body,
        grid=(_N // nb,),
        in_specs=[nspec, wspec, bspec, wspec, bspec, wspec, bspec, wspec, bspec],
        out_specs=[nspec, nspec, pl.BlockSpec((nb, 2 * _D), lambda i: (i, 0))],
        out_shape=[
            jax.ShapeDtypeStruct((_N, _D), f32),
            jax.ShapeDtypeStruct((_N, _D), f32),
            jax.ShapeDtypeStruct((_N, 2 * _D), f32),
        ],
    )(x_in_node, A_w, A_b, B_w, B_b, D_w, D_b, E_w, E_b)

    ebk = 4000
    cr = pl.pallas_call(
        _edge_proj_body,
        grid=(_E // ebk,),
        in_specs=[
            pl.BlockSpec((ebk, 16), lambda i: (i, 0)),
            pl.BlockSpec((16, _D), lambda i: (0, 0)),
            pl.BlockSpec((_D,), lambda i: (0,)),
            pl.BlockSpec((16, _D), lambda i: (0, 0)),
        ],
        out_specs=pl.BlockSpec((ebk, 2 * _D), lambda i: (i, 0)),
        out_shape=jax.ShapeDtypeStruct((_E, 2 * _D), f32),
    )(edge_in_attr, C_w, C_b, Rproj_e_w)

    rc = jnp.stack(
        [edge_idx[0].reshape(_NCH, _CHUNK), edge_idx[1].reshape(_NCH, _CHUNK)],
        axis=1,
    )
    zeros = jnp.zeros((_NPAD, _D), f32)

    mesh = plsc.VectorSubcoreMesh(core_axis_name="c", subcore_axis_name="s")
    sc_call = functools.partial(
        pl.kernel,
        out_type=(
            jax.ShapeDtypeStruct((_E, _D), f32),
            jax.ShapeDtypeStruct((2, _NPAD, _D), f32),
        ),
        mesh=mesh,
        scratch_types=[
            pltpu.VMEM((2, _CHUNK), jnp.int32),
            pltpu.VMEM((2, _CHUNK), jnp.int32),
            pltpu.VMEM((_CHUNK, _D), f32),
            pltpu.VMEM((_CHUNK, _D), f32),
            pltpu.VMEM((_CHUNK, 2 * _D), f32),
            pltpu.VMEM((_CHUNK, 2 * _D), f32),
            pltpu.VMEM((_CHUNK, 2 * _D), f32),
            pltpu.VMEM((_CHUNK, 2 * _D), f32),
            pltpu.VMEM((_CHUNK, _D), f32),
            pltpu.VMEM((_CHUNK, _D), f32),
            pltpu.SemaphoreType.DMA,
            pltpu.SemaphoreType.DMA,
            pltpu.SemaphoreType.DMA,
            pltpu.SemaphoreType.DMA,
            pltpu.SemaphoreType.DMA,
            pltpu.SemaphoreType.DMA,
            pltpu.SemaphoreType.DMA,
            pltpu.SemaphoreType.DMA,
            pltpu.SemaphoreType.DMA,
            pltpu.SemaphoreType.DMA,
            pltpu.VMEM_SHARED((_NPAD, _D), f32),
        ],
    )(_sc_edge_body)
    e_final, aggr = sc_call(rc, dx, ebt, cr, zeros)

    nb2 = 400
    x_final = pl.pallas_call(
        _tail_body,
        grid=(_N // nb2,),
        in_specs=[
            pl.BlockSpec((nb2, _D), lambda i: (i, 0)),
            pl.BlockSpec((nb2, _D), lambda i: (i, 0)),
            pl.BlockSpec((2, nb2, _D), lambda i: (0, i, 0)),
        ],
        out_specs=pl.BlockSpec((nb2, _D), lambda i: (i, 0)),
        out_shape=jax.ShapeDtypeStruct((_N, _D), f32),
    )(x_in_node, ax, aggr)

    return (x_final, e_final)
